# Initial kernel scaffold; baseline (speedup 1.0000x reference)
import jax
import jax.numpy as jnp
from jax.experimental import pallas as pl


def kernel(x, kp0, kp1, kp2, kp3, kp4, kp5, kp6, kp7, kp8, kp9):
    raise NotImplementedError("write your pallas kernel here")


# speedup vs baseline: 2.1608x; 2.1608x over previous
import functools

import jax
import jax.numpy as jnp
from jax import lax
from jax.experimental import pallas as pl
from jax.experimental.pallas import tpu as pltpu


def _sigmoid(x):
    return pl.reciprocal(1.0 + jnp.exp(-x), approx=True)


def _dot_tb(a, w):
    return lax.dot_general(a, w, (((1,), (1,)), ((), ())),
                           preferred_element_type=jnp.float32)


def _make_body(L, H, TB, F):
    def body(x_ref, wi1, wh1, bi1, bhn1, wi2, wh2, bi2, bhn2, wout, bout,
             out_ref, gi_ref, act_ref):
        def proj(t, c):
            gi_ref[t] = _dot_tb(x_ref[t], wi1[...]) + bi1[...]
            return c

        lax.fori_loop(0, L, proj, 0, unroll=8)

        bhn1v = bhn1[...]
        bhn2v = bhn2[...]
        bi2v = bi2[...]
        wh1v = wh1[...]
        wi2v = wi2[...]
        wh2v = wh2[...]

        def cell(gi, gh, bhn, h):
            r = _sigmoid(gi[:, :H] + gh[:, :H])
            z = _sigmoid(gi[:, H:2 * H] + gh[:, H:2 * H])
            n = jnp.tanh(gi[:, 2 * H:] + r * (gh[:, 2 * H:] + bhn))
            return n + z * (h - n)

        def step1(t, h1):
            gh = _dot_tb(h1.astype(jnp.bfloat16), wh1v)
            return cell(gi_ref[t], gh, bhn1v, h1)

        def step2(h1_in, h2):
            gi = _dot_tb(h1_in, wi2v) + bi2v
            gh = _dot_tb(h2.astype(jnp.bfloat16), wh2v)
            return cell(gi, gh, bhn2v, h2)

        zero_h = jnp.zeros((TB, H), jnp.float32)
        h1_0 = step1(0, zero_h)

        def chain(t, carry):
            h1_prev, h2_prev = carry
            h1b = h1_prev.astype(jnp.bfloat16)
            h1_t = cell(gi_ref[t], _dot_tb(h1b, wh1v), bhn1v, h1_prev)
            h2_t = step2(h1b, h2_prev)
            act_ref[t - 1] = h2_t.astype(jnp.bfloat16)
            return (h1_t, h2_t)

        h1_last, h2_prev = lax.fori_loop(1, L, chain, (h1_0, zero_h),
                                         unroll=4)
        h2_last = step2(h1_last.astype(jnp.bfloat16), h2_prev)
        act_ref[L - 1] = h2_last.astype(jnp.bfloat16)

        w = wout[...]
        logits = jnp.sum(act_ref[...].astype(jnp.float32) * w[None], axis=2)
        out_ref[...] = _sigmoid(logits + bout[...])

    return body


def _round_up(x, m):
    return (x + m - 1) // m * m


@jax.jit
def kernel(x, kp0, kp1, kp2, kp3, kp4, kp5, kp6, kp7, kp8, kp9):
    L, B, F = x.shape
    H = kp8.shape[0]
    TB = 256 if B % 256 == 0 else _round_up(min(B, 256), 8)
    B_pad = _round_up(B, TB)

    xb = x.astype(jnp.bfloat16)
    if B_pad != B:
        xb = jnp.pad(xb, ((0, 0), (0, B_pad - B), (0, 0)))

    wi1 = kp0.astype(jnp.bfloat16)
    wh1 = kp1.astype(jnp.bfloat16)
    bi1 = kp2.reshape(1, 3 * H)
    bhn1 = kp3.reshape(1, H)
    wi2 = kp4.astype(jnp.bfloat16)
    wh2 = kp5.astype(jnp.bfloat16)
    bi2 = kp6.reshape(1, 3 * H)
    bhn2 = kp7.reshape(1, H)
    wout = kp8.reshape(1, H)
    bout = kp9

    params = [wi1, wh1, bi1, bhn1, wi2, wh2, bi2, bhn2, wout, bout]
    w_specs = [pl.BlockSpec(p.shape, lambda i, nd=p.ndim: (0,) * nd)
               for p in params]

    grid = (B_pad // TB,)
    out = pl.pallas_call(
        _make_body(L, H, TB, F),
        out_shape=jax.ShapeDtypeStruct((L, B_pad), jnp.float32),
        grid=grid,
        in_specs=[pl.BlockSpec((L, TB, F), lambda i: (0, i, 0))] + w_specs,
        out_specs=pl.BlockSpec((L, TB), lambda i: (0, i)),
        scratch_shapes=[pltpu.VMEM((L, TB, 3 * H), jnp.float32),
                        pltpu.VMEM((L, TB, H), jnp.bfloat16)],
        compiler_params=pltpu.CompilerParams(
            dimension_semantics=("parallel",),
            vmem_limit_bytes=64 * 1024 * 1024),
    )(xb, *params)

    if B_pad != B:
        out = out[:, :B]
    return out[:, :, None]


# speedup vs baseline: 2.6951x; 1.2473x over previous
import functools

import jax
import jax.numpy as jnp
from jax import lax
from jax.experimental import pallas as pl
from jax.experimental.pallas import tpu as pltpu


def _sigmoid(x):
    return pl.reciprocal(1.0 + jnp.exp(-x), approx=True)


def _dot_tb(a, w):
    return lax.dot_general(a, w, (((1,), (1,)), ((), ())),
                           preferred_element_type=jnp.float32)


def _make_body(L, H, TB, F):
    def body(x_ref, wi1, wh1, bi1, bhn1, wi2, wh2, bi2, bhn2, wout, bout,
             out_ref, gi_ref, act_ref):
        def proj(t, c):
            gi_ref[t] = _dot_tb(x_ref[t], wi1[...]) + bi1[...]
            return c

        lax.fori_loop(0, L, proj, 0, unroll=8)

        bhn1v = bhn1[...]
        bhn2v = bhn2[...]
        bi2v = bi2[...]
        wh1_rz = wh1[:2 * H, :]
        wh1_n = wh1[2 * H:, :]
        wh2_rz = wh2[:2 * H, :]
        wh2_n = wh2[2 * H:, :]
        wi2v = wi2[...]

        def cell(gi, h, wh_rz, wh_n, bhn):
            grz = _dot_tb(h, wh_rz) + gi[:, :2 * H]
            r = _sigmoid(grz[:, :H])
            z = _sigmoid(grz[:, H:])
            ghn = _dot_tb(h, wh_n)
            n = jnp.tanh(gi[:, 2 * H:] + r * (ghn + bhn))
            return n + z * (h - n)

        def step1(t, h1):
            return cell(gi_ref[t], h1, wh1_rz, wh1_n, bhn1v)

        def step2(h1_in, h2):
            gi = _dot_tb(h1_in, wi2v) + bi2v
            return cell(gi, h2, wh2_rz, wh2_n, bhn2v)

        zero_h = jnp.zeros((TB, H), jnp.float32)
        h1_0 = step1(0, zero_h)

        def chain(t, carry):
            h1_prev, h2_prev = carry
            h1_t = step1(t, h1_prev)
            h2_t = step2(h1_prev, h2_prev)
            act_ref[t - 1] = h2_t
            return (h1_t, h2_t)

        h1_last, h2_prev = lax.fori_loop(1, L, chain, (h1_0, zero_h),
                                         unroll=4)
        h2_last = step2(h1_last, h2_prev)
        act_ref[L - 1] = h2_last

        w = wout[...]
        logits = jnp.sum(act_ref[...] * w[None], axis=2)
        out_ref[...] = _sigmoid(logits + bout[...])

    return body


def _round_up(x, m):
    return (x + m - 1) // m * m


@jax.jit
def kernel(x, kp0, kp1, kp2, kp3, kp4, kp5, kp6, kp7, kp8, kp9):
    L, B, F = x.shape
    H = kp8.shape[0]
    TB = 256 if B % 256 == 0 else _round_up(min(B, 256), 8)
    B_pad = _round_up(B, TB)

    xb = x
    if B_pad != B:
        xb = jnp.pad(xb, ((0, 0), (0, B_pad - B), (0, 0)))

    bi1 = kp2.reshape(1, 3 * H)
    bhn1 = kp3.reshape(1, H)
    bi2 = kp6.reshape(1, 3 * H)
    bhn2 = kp7.reshape(1, H)
    wout = kp8.reshape(1, H)
    bout = kp9

    params = [kp0, kp1, bi1, bhn1, kp4, kp5, bi2, bhn2, wout, bout]
    w_specs = [pl.BlockSpec(p.shape, lambda i, nd=p.ndim: (0,) * nd)
               for p in params]

    grid = (B_pad // TB,)
    out = pl.pallas_call(
        _make_body(L, H, TB, F),
        out_shape=jax.ShapeDtypeStruct((L, B_pad), jnp.float32),
        grid=grid,
        in_specs=[pl.BlockSpec((L, TB, F), lambda i: (0, i, 0))] + w_specs,
        out_specs=pl.BlockSpec((L, TB), lambda i: (0, i)),
        scratch_shapes=[pltpu.VMEM((L, TB, 3 * H), jnp.float32),
                        pltpu.VMEM((L, TB, H), jnp.float32)],
        compiler_params=pltpu.CompilerParams(
            dimension_semantics=("parallel",),
            vmem_limit_bytes=64 * 1024 * 1024),
    )(xb, *params)

    if B_pad != B:
        out = out[:, :B]
    return out[:, :, None]


# speedup vs baseline: 3.1306x; 1.1616x over previous
import functools

import jax
import jax.numpy as jnp
from jax import lax
from jax.experimental import pallas as pl
from jax.experimental.pallas import tpu as pltpu


def _sigmoid(x):
    return pl.reciprocal(1.0 + jnp.exp(-x), approx=True)


def _dot_tb(a, w):
    return lax.dot_general(a, w, (((1,), (1,)), ((), ())),
                           preferred_element_type=jnp.float32)


def _make_body(L, H, TB, F):
    def body(x_ref, wi1, wh1, bi1, bhn1, wi2, wh2, bi2, bhn2, wout, bout,
             out_ref, gi_ref, act_ref):
        def proj(t, c):
            gi_ref[t] = _dot_tb(x_ref[t], wi1[...]) + bi1[...]
            return c

        lax.fori_loop(0, L, proj, 0, unroll=8)

        bhn1v = bhn1[...]
        bhn2v = bhn2[...]
        bi2v = bi2[...]
        wh1_rz = wh1[:2 * H, :]
        wh1_n = wh1[2 * H:, :]
        wh2_rz = wh2[:2 * H, :]
        wh2_n = wh2[2 * H:, :]
        wi2v = wi2[...]

        def cell(gi, h, wh_rz, wh_n, bhn):
            grz = _dot_tb(h, wh_rz) + gi[:, :2 * H]
            r = _sigmoid(grz[:, :H])
            z = _sigmoid(grz[:, H:])
            ghn = _dot_tb(h, wh_n)
            n = jnp.tanh(gi[:, 2 * H:] + r * (ghn + bhn))
            return n + z * (h - n)

        def step1(t, h1):
            return cell(gi_ref[t], h1, wh1_rz, wh1_n, bhn1v)

        def step2(h1_in, h2):
            gi = _dot_tb(h1_in, wi2v) + bi2v
            return cell(gi, h2, wh2_rz, wh2_n, bhn2v)

        zero_h = jnp.zeros((TB, H), jnp.float32)
        h1_0 = step1(0, zero_h)

        def chain(t, carry):
            h1_prev, h2_prev = carry
            h1_t = step1(t, h1_prev)
            h2_t = step2(h1_prev, h2_prev)
            act_ref[t - 1] = h2_t
            return (h1_t, h2_t)

        h1_last, h2_prev = lax.fori_loop(1, L, chain, (h1_0, zero_h),
                                         unroll=31)
        h2_last = step2(h1_last, h2_prev)
        act_ref[L - 1] = h2_last

        w = wout[...]
        logits = jnp.sum(act_ref[...] * w[None], axis=2)
        out_ref[...] = _sigmoid(logits + bout[...])

    return body


def _round_up(x, m):
    return (x + m - 1) // m * m


@jax.jit
def kernel(x, kp0, kp1, kp2, kp3, kp4, kp5, kp6, kp7, kp8, kp9):
    L, B, F = x.shape
    H = kp8.shape[0]
    TB = 256 if B % 256 == 0 else _round_up(min(B, 256), 8)
    B_pad = _round_up(B, TB)

    xb = x
    if B_pad != B:
        xb = jnp.pad(xb, ((0, 0), (0, B_pad - B), (0, 0)))

    bi1 = kp2.reshape(1, 3 * H)
    bhn1 = kp3.reshape(1, H)
    bi2 = kp6.reshape(1, 3 * H)
    bhn2 = kp7.reshape(1, H)
    wout = kp8.reshape(1, H)
    bout = kp9

    params = [kp0, kp1, bi1, bhn1, kp4, kp5, bi2, bhn2, wout, bout]
    w_specs = [pl.BlockSpec(p.shape, lambda i, nd=p.ndim: (0,) * nd)
               for p in params]

    grid = (B_pad // TB,)
    out = pl.pallas_call(
        _make_body(L, H, TB, F),
        out_shape=jax.ShapeDtypeStruct((L, B_pad), jnp.float32),
        grid=grid,
        in_specs=[pl.BlockSpec((L, TB, F), lambda i: (0, i, 0))] + w_specs,
        out_specs=pl.BlockSpec((L, TB), lambda i: (0, i)),
        scratch_shapes=[pltpu.VMEM((L, TB, 3 * H), jnp.float32),
                        pltpu.VMEM((L, TB, H), jnp.float32)],
        compiler_params=pltpu.CompilerParams(
            dimension_semantics=("parallel",),
            vmem_limit_bytes=64 * 1024 * 1024),
    )(xb, *params)

    if B_pad != B:
        out = out[:, :B]
    return out[:, :, None]


# speedup vs baseline: 3.9840x; 1.2726x over previous
import functools

import jax
import jax.numpy as jnp
from jax import lax
from jax.experimental import pallas as pl
from jax.experimental.pallas import tpu as pltpu


def _sigmoid(x):
    return pl.reciprocal(1.0 + jnp.exp(-x), approx=True)


def _dot_tb(a, w):
    return lax.dot_general(a, w, (((1,), (1,)), ((), ())),
                           preferred_element_type=jnp.float32)


def _make_body(L, H, TB, F):
    def body(x_ref, wi1, wh1, bi1, bhn1, wi2, wh2, bi2, bhn2, wout, bout,
             out_ref, act_ref):
        bi1v = bi1[...]
        bhn1v = bhn1[...]
        bhn2v = bhn2[...]
        bi2v = bi2[...]
        wi1v = wi1[...]
        wh1_rz = wh1[:2 * H, :]
        wh1_n = wh1[2 * H:, :]
        wh2_rz = wh2[:2 * H, :]
        wh2_n = wh2[2 * H:, :]
        wi2v = wi2[...]

        def cell(gi, h, wh_rz, wh_n, bhn):
            grz = _dot_tb(h, wh_rz) + gi[:, :2 * H]
            r = _sigmoid(grz[:, :H])
            z = _sigmoid(grz[:, H:])
            ghn = _dot_tb(h, wh_n)
            n = jnp.tanh(gi[:, 2 * H:] + r * (ghn + bhn))
            return n + z * (h - n)

        def step1(t, h1):
            gi = _dot_tb(x_ref[t], wi1v) + bi1v
            return cell(gi, h1, wh1_rz, wh1_n, bhn1v)

        def step2(h1_in, h2):
            gi = _dot_tb(h1_in, wi2v) + bi2v
            return cell(gi, h2, wh2_rz, wh2_n, bhn2v)

        zero_h = jnp.zeros((TB, H), jnp.float32)
        h1 = step1(0, zero_h)
        h2 = zero_h
        for t in range(1, L):
            h1_next = step1(t, h1)
            h2 = step2(h1, h2)
            act_ref[t - 1] = h2
            h1 = h1_next
        h2 = step2(h1, h2)
        act_ref[L - 1] = h2

        w = wout[...]
        logits = jnp.sum(act_ref[...] * w[None], axis=2)
        out_ref[...] = _sigmoid(logits + bout[...])

    return body


def _round_up(x, m):
    return (x + m - 1) // m * m


@jax.jit
def kernel(x, kp0, kp1, kp2, kp3, kp4, kp5, kp6, kp7, kp8, kp9):
    L, B, F = x.shape
    H = kp8.shape[0]
    TB = 256 if B % 256 == 0 else _round_up(min(B, 256), 8)
    B_pad = _round_up(B, TB)

    xb = x
    if B_pad != B:
        xb = jnp.pad(xb, ((0, 0), (0, B_pad - B), (0, 0)))

    bi1 = kp2.reshape(1, 3 * H)
    bhn1 = kp3.reshape(1, H)
    bi2 = kp6.reshape(1, 3 * H)
    bhn2 = kp7.reshape(1, H)
    wout = kp8.reshape(1, H)
    bout = kp9

    params = [kp0, kp1, bi1, bhn1, kp4, kp5, bi2, bhn2, wout, bout]
    w_specs = [pl.BlockSpec(p.shape, lambda i, nd=p.ndim: (0,) * nd)
               for p in params]

    grid = (B_pad // TB,)
    out = pl.pallas_call(
        _make_body(L, H, TB, F),
        out_shape=jax.ShapeDtypeStruct((L, B_pad), jnp.float32),
        grid=grid,
        in_specs=[pl.BlockSpec((L, TB, F), lambda i: (0, i, 0))] + w_specs,
        out_specs=pl.BlockSpec((L, TB), lambda i: (0, i)),
        scratch_shapes=[pltpu.VMEM((L, TB, H), jnp.float32)],
        compiler_params=pltpu.CompilerParams(
            dimension_semantics=("parallel",),
            vmem_limit_bytes=64 * 1024 * 1024),
    )(xb, *params)

    if B_pad != B:
        out = out[:, :B]
    return out[:, :, None]
